# baseline (speedup 1.0000x reference)
import functools

import jax
import jax.numpy as jnp
from jax import lax
from jax.experimental import pallas as pl
from jax.experimental.pallas import tpu as pltpu
from jax.experimental.pallas import tpu_sc as plsc

N = 10000
E = 320000
D = 128
H = 128
G = 8

NC = 2
NS = 16
NW = NC * NS

CHUNK = 128
ROWS_TOTAL = E // CHUNK
ROWS_PER_TILE = -(-ROWS_TOTAL // (NW * 8)) * 8
ROWS_PAD = ROWS_PER_TILE * NW
SEG = 40
FAST_ROWS = 120
SLOW_ROWS = ROWS_PAD // NS - FAST_ROWS
DW = 64
ACC_ROWS = 10240
ZROWS = ACC_ROWS // NS
TRASH = N

_mesh = plsc.VectorSubcoreMesh(core_axis_name="c", subcore_axis_name="s")


@functools.partial(
    pl.kernel,
    out_type=jax.ShapeDtypeStruct((NC, ACC_ROWS, DW), jnp.float32),
    mesh=_mesh,
    scratch_types=[
        pltpu.VMEM((ROWS_PER_TILE, CHUNK), jnp.int32),
        pltpu.VMEM((CHUNK, DW), jnp.float32),
        pltpu.VMEM((16, DW), jnp.float32),
        pltpu.VMEM_SHARED((ACC_ROWS, DW), jnp.float32),
        pltpu.SemaphoreType.DMA,
    ],
)
def _sc_degree(dst_hbm, out_hbm, dst_v, ones_v, zbuf, acc, sem):
    c = lax.axis_index("c")
    s = lax.axis_index("s")
    wid = s * NC + c

    @pl.loop(0, 16)
    def _(i):
        @pl.loop(0, DW, step=16)
        def _(j):
            zbuf[i, pl.ds(j, 16)] = jnp.zeros((16,), jnp.float32)

    @pl.loop(0, CHUNK)
    def _(i):
        @pl.loop(0, DW, step=16)
        def _(j):
            ones_v[i, pl.ds(j, 16)] = jnp.ones((16,), jnp.float32)

    @pl.loop(0, ZROWS, step=16)
    def _(r):
        pltpu.sync_copy(zbuf, acc.at[pl.ds(s * ZROWS + r, 16)])

    pltpu.async_copy(
        dst_hbm.at[pl.ds(wid * ROWS_PER_TILE, ROWS_PER_TILE)], dst_v, sem
    ).wait()
    plsc.subcore_barrier()

    @pl.loop(0, ROWS_PER_TILE, step=8)
    def _(j):
        for b in range(8):
            pltpu.async_copy(ones_v, acc.at[dst_v.at[j + b]], sem, add=True)
        for b in range(8):
            pltpu.make_async_copy(ones_v, acc.at[dst_v.at[j + b]], sem).wait()

    plsc.subcore_barrier()
    pltpu.sync_copy(
        acc.at[pl.ds(s * ZROWS, ZROWS)],
        out_hbm.at[c].at[pl.ds(s * ZROWS, ZROWS)],
    )


@functools.partial(
    pl.kernel,
    out_type=jax.ShapeDtypeStruct((NC, ACC_ROWS, D), jnp.float32),
    mesh=_mesh,
    scratch_types=[
        pltpu.VMEM((SEG, CHUNK), jnp.int32),
        pltpu.VMEM((SEG, CHUNK), jnp.int32),
        pltpu.VMEM((CHUNK, D), jnp.float32),
        pltpu.VMEM((CHUNK, D), jnp.float32),
        pltpu.VMEM((16, D), jnp.float32),
        pltpu.VMEM_SHARED((ACC_ROWS, D), jnp.float32),
        pltpu.SemaphoreType.DMA,
        pltpu.SemaphoreType.DMA,
        pltpu.SemaphoreType.DMA,
    ],
)
def _sc_aggregate(xs_hbm, src_hbm, dst_hbm, out_hbm,
                  src_v, dst_v, rows0, rows1, zbuf, acc, sem, gsem0, gsem1):
    c = lax.axis_index("c")
    s = lax.axis_index("s")
    fast = c == 0
    ptile = jnp.where(fast, FAST_ROWS, SLOW_ROWS)
    c_off = jnp.where(fast, 0, NS * FAST_ROWS)
    nseg = jnp.where(fast, FAST_ROWS // SEG, SLOW_ROWS // SEG)
    tile_base = c_off + s * ptile

    @pl.loop(0, 16)
    def _(i):
        @pl.loop(0, D, step=16)
        def _(j):
            zbuf[i, pl.ds(j, 16)] = jnp.zeros((16,), jnp.float32)

    @pl.loop(0, ZROWS, step=16)
    def _(r):
        pltpu.sync_copy(zbuf, acc.at[pl.ds(s * ZROWS + r, 16)])

    plsc.subcore_barrier()

    @pl.loop(0, nseg)
    def _(t):
        seg = tile_base + t * SEG
        pltpu.async_copy(src_hbm.at[pl.ds(seg, SEG)], src_v, sem).wait()
        pltpu.async_copy(dst_hbm.at[pl.ds(seg, SEG)], dst_v, sem).wait()

        pltpu.async_copy(xs_hbm.at[src_v.at[0]], rows0, gsem0)

        @pl.loop(0, SEG, step=2)
        def _(j):
            pltpu.async_copy(xs_hbm.at[src_v.at[j + 1]], rows1, gsem1)
            pltpu.make_async_copy(xs_hbm.at[src_v.at[j]], rows0, gsem0).wait()
            pltpu.sync_copy(rows0, acc.at[dst_v.at[j]], add=True)

            @pl.when(j + 2 < SEG)
            def _():
                pltpu.async_copy(xs_hbm.at[src_v.at[j + 2]], rows0, gsem0)

            pltpu.make_async_copy(
                xs_hbm.at[src_v.at[j + 1]], rows1, gsem1).wait()
            pltpu.sync_copy(rows1, acc.at[dst_v.at[j + 1]], add=True)

    plsc.subcore_barrier()
    pltpu.sync_copy(
        acc.at[pl.ds(s * ZROWS, ZROWS)],
        out_hbm.at[c].at[pl.ds(s * ZROWS, ZROWS)],
    )


def _tc0_body(x_ref, w_ref, xw_ref):
    xw_ref[...] = jnp.dot(x_ref[...], w_ref[...],
                          preferred_element_type=jnp.float32)


def _tc1_body(xw_ref, degp_ref, xs_ref, dis_ref):
    dp = degp_ref[...]
    deg = 1.0 + dp[0, :N, 0:1] + dp[1, :N, 0:1]
    dis = lax.rsqrt(deg)
    xs_ref[...] = xw_ref[...] * dis
    dis_ref[...] = dis


def _tc2_body(p_ref, xs_ref, dis_ref, b_ref, w_ref, out_ref):
    dis = dis_ref[...]
    p = p_ref[...]
    h = dis * (p[0, :N] + p[1, :N] + xs_ref[...]) + b_ref[...]
    h = jnp.maximum(h, 0.0)
    xw = jnp.dot(h, w_ref[...], preferred_element_type=jnp.float32)
    out_ref[...] = xw * dis


def _tc3_body(p_ref, xs_ref, dis_ref, b_ref, batch_ref,
              wp_ref, bp_ref, out_ref):
    dis = dis_ref[...]
    p = p_ref[...]
    h = dis * (p[0, :N] + p[1, :N] + xs_ref[...]) + b_ref[...]
    h = jnp.maximum(h, 0.0)
    gid = lax.broadcasted_iota(jnp.int32, (N, G), 1)
    oh = (batch_ref[...] == gid).astype(jnp.float32)
    sums = lax.dot_general(oh, h, (((0,), (0,)), ((), ())),
                           preferred_element_type=jnp.float32,
                           precision=lax.Precision.HIGHEST)
    counts = lax.dot_general(oh, jnp.ones((N, 1), jnp.float32),
                             (((0,), (0,)), ((), ())),
                             preferred_element_type=jnp.float32,
                             precision=lax.Precision.HIGHEST)
    pooled = sums / jnp.maximum(counts, 1.0)
    out_ref[...] = jnp.dot(pooled, wp_ref[...],
                           preferred_element_type=jnp.float32) + bp_ref[...]


_tc0 = pl.pallas_call(
    _tc0_body,
    out_shape=jax.ShapeDtypeStruct((N, D), jnp.float32),
)

_tc1 = pl.pallas_call(
    _tc1_body,
    out_shape=[jax.ShapeDtypeStruct((N, D), jnp.float32),
               jax.ShapeDtypeStruct((N, 1), jnp.float32)],
)

_tc2 = pl.pallas_call(
    _tc2_body,
    out_shape=jax.ShapeDtypeStruct((N, H), jnp.float32),
)

_tc3 = pl.pallas_call(
    _tc3_body,
    out_shape=jax.ShapeDtypeStruct((G, 1), jnp.float32),
)


def kernel(x, edge_index, batch, W1, b1, W2, b2, Wp, bp):
    ei = edge_index.astype(jnp.int32)
    pad = ROWS_PAD * CHUNK - E
    src2d = jnp.concatenate(
        [ei[0], jnp.zeros((pad,), jnp.int32)]).reshape(ROWS_PAD, CHUNK)
    dst2d = jnp.concatenate(
        [ei[1], jnp.full((pad,), TRASH, jnp.int32)]).reshape(ROWS_PAD, CHUNK)
    batch2d = batch.astype(jnp.int32).reshape(N, 1)
    b1r = b1.reshape(1, H)
    b2r = b2.reshape(1, H)
    bpr = bp.reshape(1, 1)

    xw1 = _tc0(x, W1)
    degp = _sc_degree(dst2d)
    xs1, dis = _tc1(xw1, degp)

    p1 = _sc_aggregate(xs1, src2d, dst2d)
    xs2 = _tc2(p1, xs1, dis, b1r, W2)

    p2 = _sc_aggregate(xs2, src2d, dst2d)
    out = _tc3(p2, xs2, dis, b2r, batch2d, Wp, bpr)

    return out.reshape(G)


# speedup vs baseline: 1.9418x; 1.9418x over previous
import functools

import jax
import jax.numpy as jnp
from jax import lax
from jax.experimental import pallas as pl
from jax.experimental.pallas import tpu as pltpu
from jax.experimental.pallas import tpu_sc as plsc

N = 10000
E = 320000
D = 128
H = 128
G = 8

NC = 2
NS = 16
NW = NC * NS

CHUNK = 128
ROWS_TOTAL = E // CHUNK
ROWS_PER_TILE = -(-ROWS_TOTAL // (NW * 8)) * 8
ROWS_PAD = ROWS_PER_TILE * NW
SEG = 40
FAST_ROWS = 120
SLOW_ROWS = ROWS_PAD // NS - FAST_ROWS
DW = 64
ACC_ROWS = 10240
ZROWS = ACC_ROWS // NS
TRASH = N

_mesh = plsc.VectorSubcoreMesh(core_axis_name="c", subcore_axis_name="s")


@functools.partial(
    pl.kernel,
    out_type=jax.ShapeDtypeStruct((NC, ACC_ROWS, DW), jnp.float32),
    mesh=_mesh,
    scratch_types=[
        pltpu.VMEM((ROWS_PER_TILE, CHUNK), jnp.int32),
        pltpu.VMEM((CHUNK, DW), jnp.float32),
        pltpu.VMEM((16, DW), jnp.float32),
        pltpu.VMEM_SHARED((ACC_ROWS, DW), jnp.float32),
        pltpu.SemaphoreType.DMA,
    ],
)
def _sc_degree(dst_hbm, out_hbm, dst_v, ones_v, zbuf, acc, sem):
    c = lax.axis_index("c")
    s = lax.axis_index("s")
    wid = s * NC + c

    @pl.loop(0, 16)
    def _(i):
        @pl.loop(0, DW, step=16)
        def _(j):
            zbuf[i, pl.ds(j, 16)] = jnp.zeros((16,), jnp.float32)

    @pl.loop(0, CHUNK)
    def _(i):
        @pl.loop(0, DW, step=16)
        def _(j):
            ones_v[i, pl.ds(j, 16)] = jnp.ones((16,), jnp.float32)

    @pl.loop(0, ZROWS, step=16)
    def _(r):
        pltpu.sync_copy(zbuf, acc.at[pl.ds(s * ZROWS + r, 16)])

    pltpu.async_copy(
        dst_hbm.at[pl.ds(wid * ROWS_PER_TILE, ROWS_PER_TILE)], dst_v, sem
    ).wait()
    plsc.subcore_barrier()

    @pl.loop(0, ROWS_PER_TILE, step=8)
    def _(j):
        for b in range(8):
            pltpu.async_copy(ones_v, acc.at[dst_v.at[j + b]], sem, add=True)
        for b in range(8):
            pltpu.make_async_copy(ones_v, acc.at[dst_v.at[j + b]], sem).wait()

    plsc.subcore_barrier()
    pltpu.sync_copy(
        acc.at[pl.ds(s * ZROWS, ZROWS)],
        out_hbm.at[c].at[pl.ds(s * ZROWS, ZROWS)],
    )


@functools.partial(
    pl.kernel,
    out_type=jax.ShapeDtypeStruct((NC, ACC_ROWS, D), jnp.float32),
    mesh=_mesh,
    scratch_types=[
        pltpu.VMEM((SEG, CHUNK), jnp.int32),
        pltpu.VMEM((SEG, CHUNK), jnp.int32),
        pltpu.VMEM((CHUNK, D), jnp.float32),
        pltpu.VMEM((CHUNK, D), jnp.float32),
        pltpu.VMEM((16, D), jnp.float32),
        pltpu.VMEM_SHARED((ACC_ROWS, D), jnp.float32),
        pltpu.SemaphoreType.DMA,
        pltpu.SemaphoreType.DMA,
        pltpu.SemaphoreType.DMA,
    ],
)
def _sc_aggregate(xs_hbm, src_hbm, dst_hbm, out_hbm,
                  src_v, dst_v, rows0, rows1, zbuf, acc, sem, gsem0, gsem1):
    c = lax.axis_index("c")
    s = lax.axis_index("s")
    fast = c == 0
    ptile = jnp.where(fast, FAST_ROWS, SLOW_ROWS)
    c_off = jnp.where(fast, 0, NS * FAST_ROWS)
    nseg = jnp.where(fast, FAST_ROWS // SEG, SLOW_ROWS // SEG)
    tile_base = c_off + s * ptile

    @pl.loop(0, 16)
    def _(i):
        @pl.loop(0, D, step=16)
        def _(j):
            zbuf[i, pl.ds(j, 16)] = jnp.zeros((16,), jnp.float32)

    @pl.loop(0, ZROWS, step=16)
    def _(r):
        pltpu.sync_copy(zbuf, acc.at[pl.ds(s * ZROWS + r, 16)])

    plsc.subcore_barrier()

    @pl.loop(0, nseg)
    def _(t):
        seg = tile_base + t * SEG
        pltpu.async_copy(src_hbm.at[pl.ds(seg, SEG)], src_v, sem).wait()
        pltpu.async_copy(dst_hbm.at[pl.ds(seg, SEG)], dst_v, sem).wait()

        pltpu.async_copy(xs_hbm.at[src_v.at[0]], rows0, gsem0)

        @pl.loop(0, SEG, step=2)
        def _(j):
            pltpu.async_copy(xs_hbm.at[src_v.at[j + 1]], rows1, gsem1)
            pltpu.make_async_copy(xs_hbm.at[src_v.at[j]], rows0, gsem0).wait()
            pltpu.sync_copy(rows0, acc.at[dst_v.at[j]], add=True)

            @pl.when(j + 2 < SEG)
            def _():
                pltpu.async_copy(xs_hbm.at[src_v.at[j + 2]], rows0, gsem0)

            pltpu.make_async_copy(
                xs_hbm.at[src_v.at[j + 1]], rows1, gsem1).wait()
            pltpu.sync_copy(rows1, acc.at[dst_v.at[j + 1]], add=True)

    plsc.subcore_barrier()
    pltpu.sync_copy(
        acc.at[pl.ds(s * ZROWS, ZROWS)],
        out_hbm.at[c].at[pl.ds(s * ZROWS, ZROWS)],
    )


def _tc0_body(x_ref, w_ref, xw_ref):
    xw_ref[...] = jnp.dot(x_ref[...], w_ref[...],
                          preferred_element_type=jnp.float32)


def _tc1_body(xw_ref, d0_ref, d1_ref, xs_ref, dis_ref):
    deg = 1.0 + d0_ref[...][:, 0:1] + d1_ref[...][:, 0:1]
    dis = lax.rsqrt(deg)
    xs_ref[...] = xw_ref[...] * dis
    dis_ref[...] = dis


def _tc2_body(p0_ref, p1_ref, xs_ref, dis_ref, b_ref, w_ref, out_ref):
    dis = dis_ref[...]
    h = dis * (p0_ref[...] + p1_ref[...] + xs_ref[...]) + b_ref[...]
    h = jnp.maximum(h, 0.0)
    xw = jnp.dot(h, w_ref[...], preferred_element_type=jnp.float32)
    out_ref[...] = xw * dis


def _tc3_body(p0_ref, p1_ref, xs_ref, dis_ref, b_ref, batch_ref,
              wp_ref, bp_ref, out_ref):
    dis = dis_ref[...]
    h = dis * (p0_ref[...] + p1_ref[...] + xs_ref[...]) + b_ref[...]
    h = jnp.maximum(h, 0.0)
    gid = lax.broadcasted_iota(jnp.int32, (N, G), 1)
    oh = (batch_ref[...] == gid).astype(jnp.float32)
    sums = lax.dot_general(oh, h, (((0,), (0,)), ((), ())),
                           preferred_element_type=jnp.float32,
                           precision=lax.Precision.HIGHEST)
    counts = lax.dot_general(oh, jnp.ones((N, 1), jnp.float32),
                             (((0,), (0,)), ((), ())),
                             preferred_element_type=jnp.float32,
                             precision=lax.Precision.HIGHEST)
    pooled = sums / jnp.maximum(counts, 1.0)
    out_ref[...] = jnp.dot(pooled, wp_ref[...],
                           preferred_element_type=jnp.float32) + bp_ref[...]


_tc0 = pl.pallas_call(
    _tc0_body,
    out_shape=jax.ShapeDtypeStruct((N, D), jnp.float32),
)

_tc1 = pl.pallas_call(
    _tc1_body,
    out_shape=[jax.ShapeDtypeStruct((N, D), jnp.float32),
               jax.ShapeDtypeStruct((N, 1), jnp.float32)],
)

_tc2 = pl.pallas_call(
    _tc2_body,
    out_shape=jax.ShapeDtypeStruct((N, H), jnp.float32),
)

_tc3 = pl.pallas_call(
    _tc3_body,
    out_shape=jax.ShapeDtypeStruct((G, 1), jnp.float32),
)


def kernel(x, edge_index, batch, W1, b1, W2, b2, Wp, bp):
    ei = edge_index.astype(jnp.int32)
    pad = ROWS_PAD * CHUNK - E
    src2d = jnp.concatenate(
        [ei[0], jnp.zeros((pad,), jnp.int32)]).reshape(ROWS_PAD, CHUNK)
    dst2d = jnp.concatenate(
        [ei[1], jnp.full((pad,), TRASH, jnp.int32)]).reshape(ROWS_PAD, CHUNK)
    batch2d = batch.astype(jnp.int32).reshape(N, 1)
    b1r = b1.reshape(1, H)
    b2r = b2.reshape(1, H)
    bpr = bp.reshape(1, 1)

    xw1 = _tc0(x, W1)
    degp = _sc_degree(dst2d)
    d0 = degp[0, :N, 0:1]
    d1 = degp[1, :N, 0:1]

    xs1, dis = _tc1(xw1, d0, d1)

    p1 = _sc_aggregate(xs1, src2d, dst2d)
    xs2 = _tc2(p1[0, :N], p1[1, :N], xs1, dis, b1r, W2)

    p2 = _sc_aggregate(xs2, src2d, dst2d)
    out = _tc3(p2[0, :N], p2[1, :N], xs2, dis, b2r, batch2d, Wp, bpr)

    return out.reshape(G)
